# baseline (speedup 1.0000x reference)
import functools

import jax
import jax.numpy as jnp
from jax import lax
from jax.experimental import pallas as pl
from jax.experimental.pallas import tpu as pltpu


def _xw_body(x_ref, w_ref, o_ref):
    o_ref[...] = jnp.dot(x_ref[...], w_ref[...], preferred_element_type=jnp.float32)


def _fused_body(adj_ref, a_ref, b1_ref, w2_ref, b2_ref, wf1t_ref, bf1_ref,
                wf2t_ref, bf2_ref, o_ref, g_ref, *, nm, bm):
    i = pl.program_id(0)

    @pl.when(i < nm)
    def _():
        h = jnp.maximum(
            jnp.dot(adj_ref[...], a_ref[...],
                    preferred_element_type=jnp.float32) + b1_ref[...], 0.0)
        g_ref[pl.ds(lax.rem(i, nm) * bm, bm), :] = jnp.dot(
            h, w2_ref[...], preferred_element_type=jnp.float32)
        o_ref[...] = jnp.zeros_like(o_ref)

    @pl.when(i >= nm)
    def _():
        emb = jnp.dot(adj_ref[...], g_ref[...],
                      preferred_element_type=jnp.float32) + b2_ref[...]
        y = jnp.maximum(
            jnp.dot(emb, wf1t_ref[...], preferred_element_type=jnp.float32)
            + bf1_ref[...], 0.0)
        y = jnp.dot(y, wf2t_ref[...],
                    preferred_element_type=jnp.float32) + bf2_ref[...]
        m = jnp.max(y, axis=1, keepdims=True)
        lse = m + jnp.log(jnp.sum(jnp.exp(y - m), axis=1, keepdims=True))
        o_ref[...] = y - lse


def kernel(x, adj, walks, W1, b1, W2, b2, W_ih, W_hh, b_ih, b_hh,
           Wf1, bf1, Wf2, bf2):
    del walks, W_ih, W_hh, b_ih, b_hh
    N, F = x.shape
    H = W1.shape[1]
    E = W2.shape[1]
    C = Wf2.shape[0]
    BM = 400
    nm = N // BM

    b1r = b1.reshape(1, H)
    b2r = b2.reshape(1, E)
    bf1r = bf1.reshape(1, -1)
    bf2r = bf2.reshape(1, C)
    wf1t = Wf1.T
    wf2t = Wf2.T

    a = pl.pallas_call(
        _xw_body,
        grid=(N // 2000,),
        in_specs=[
            pl.BlockSpec((2000, F), lambda i: (i, 0)),
            pl.BlockSpec((F, H), lambda i: (0, 0)),
        ],
        out_specs=pl.BlockSpec((2000, H), lambda i: (i, 0)),
        out_shape=jax.ShapeDtypeStruct((N, H), jnp.float32),
    )(x, W1)

    blk = lambda i: (lax.rem(i, nm), 0)
    const = lambda i: (0, 0)
    out = pl.pallas_call(
        functools.partial(_fused_body, nm=nm, bm=BM),
        grid=(2 * nm,),
        in_specs=[
            pl.BlockSpec((BM, N), blk),
            pl.BlockSpec((N, H), const),
            pl.BlockSpec((1, H), const),
            pl.BlockSpec((H, E), const),
            pl.BlockSpec((1, E), const),
            pl.BlockSpec((E, wf1t.shape[1]), const),
            pl.BlockSpec((1, bf1r.shape[1]), const),
            pl.BlockSpec((wf2t.shape[0], C), const),
            pl.BlockSpec((1, C), const),
        ],
        out_specs=pl.BlockSpec((BM, C), blk),
        out_shape=jax.ShapeDtypeStruct((N, C), jnp.float32),
        scratch_shapes=[pltpu.VMEM((N, E), jnp.float32)],
        compiler_params=pltpu.CompilerParams(
            dimension_semantics=("arbitrary",)),
    )(adj, a, b1r, W2, b2r, wf1t, bf1r, wf2t, bf2r)

    return out


# speedup vs baseline: 1.0278x; 1.0278x over previous
import functools

import jax
import jax.numpy as jnp
from jax import lax
from jax.experimental import pallas as pl
from jax.experimental.pallas import tpu as pltpu


def _fused_body(adj_ref, x_ref, w1_ref, b1_ref, w2_ref, b2_ref, wf1t_ref,
                bf1_ref, wf2t_ref, bf2_ref, o_ref, a_ref, g_ref, *, nm, bm):
    i = pl.program_id(0)

    @pl.when(i == 0)
    def _():
        a_ref[...] = jnp.dot(x_ref[...], w1_ref[...],
                             preferred_element_type=jnp.float32)

    @pl.when(i < nm)
    def _():
        h = jnp.maximum(
            jnp.dot(adj_ref[...], a_ref[...],
                    preferred_element_type=jnp.float32) + b1_ref[...], 0.0)
        g_ref[pl.ds(lax.rem(i, nm) * bm, bm), :] = jnp.dot(
            h, w2_ref[...], preferred_element_type=jnp.float32)
        o_ref[...] = jnp.zeros_like(o_ref)

    @pl.when(i >= nm)
    def _():
        emb = jnp.dot(adj_ref[...], g_ref[...],
                      preferred_element_type=jnp.float32) + b2_ref[...]
        y = jnp.maximum(
            jnp.dot(emb, wf1t_ref[...], preferred_element_type=jnp.float32)
            + bf1_ref[...], 0.0)
        y = jnp.dot(y, wf2t_ref[...],
                    preferred_element_type=jnp.float32) + bf2_ref[...]
        m = jnp.max(y, axis=1, keepdims=True)
        lse = m + jnp.log(jnp.sum(jnp.exp(y - m), axis=1, keepdims=True))
        o_ref[...] = y - lse


def kernel(x, adj, walks, W1, b1, W2, b2, W_ih, W_hh, b_ih, b_hh,
           Wf1, bf1, Wf2, bf2):
    del walks, W_ih, W_hh, b_ih, b_hh
    N, F = x.shape
    H = W1.shape[1]
    E = W2.shape[1]
    C = Wf2.shape[0]
    BM = 400
    nm = N // BM

    b1r = b1.reshape(1, H)
    b2r = b2.reshape(1, E)
    bf1r = bf1.reshape(1, -1)
    bf2r = bf2.reshape(1, C)
    wf1t = Wf1.T
    wf2t = Wf2.T

    blk = lambda i: (lax.rem(i, nm), 0)
    const = lambda i: (0, 0)
    out = pl.pallas_call(
        functools.partial(_fused_body, nm=nm, bm=BM),
        grid=(2 * nm,),
        in_specs=[
            pl.BlockSpec((BM, N), blk),
            pl.BlockSpec((N, F), const),
            pl.BlockSpec((F, H), const),
            pl.BlockSpec((1, H), const),
            pl.BlockSpec((H, E), const),
            pl.BlockSpec((1, E), const),
            pl.BlockSpec((E, wf1t.shape[1]), const),
            pl.BlockSpec((1, bf1r.shape[1]), const),
            pl.BlockSpec((wf2t.shape[0], C), const),
            pl.BlockSpec((1, C), const),
        ],
        out_specs=pl.BlockSpec((BM, C), blk),
        out_shape=jax.ShapeDtypeStruct((N, C), jnp.float32),
        scratch_shapes=[pltpu.VMEM((N, H), jnp.float32),
                        pltpu.VMEM((N, E), jnp.float32)],
        compiler_params=pltpu.CompilerParams(
            dimension_semantics=("arbitrary",)),
    )(adj, x, W1, b1r, W2, b2r, wf1t, bf1r, wf2t, bf2r)

    return out
